# baseline (speedup 1.0000x reference)
import functools

import jax
import jax.numpy as jnp
from jax.experimental import pallas as pl
from jax.experimental.pallas import tpu as pltpu

RHO = 0.5
TAU = 0.99
N_SINKHORN = 50
EPS = 1e-8


def _prologue_body(logits_ref, py_ref, pred_ref, cost_ref, logsm_ref, rn_ref):
    l = logits_ref[:, :]
    py = py_ref[:, :]
    m = jnp.max(l, axis=1, keepdims=True)
    ex = jnp.exp(l - m)
    s = jnp.sum(ex, axis=1, keepdims=True)
    pred = ex / s
    logsm = (l - m) - jnp.log(s)
    conf = py / jnp.sum(py, axis=1, keepdims=True)
    pred_ref[:, :] = pred
    cost_ref[:, :] = pred * py
    logsm_ref[:, :] = logsm
    rn_ref[:, :] = -jnp.sum(logsm * conf, axis=1, keepdims=True)


def _prologue(logits_w, partial_y, tb=512):
    b, c = logits_w.shape
    grid = (b // tb,)
    bs = pl.BlockSpec((tb, c), lambda i: (i, 0))
    vs = pl.BlockSpec((tb, 1), lambda i: (i, 0))
    return pl.pallas_call(
        _prologue_body,
        grid=grid,
        in_specs=[bs, bs],
        out_specs=[bs, bs, bs, vs],
        out_shape=[
            jax.ShapeDtypeStruct((b, c), jnp.float32),
            jax.ShapeDtypeStruct((b, c), jnp.float32),
            jax.ShapeDtypeStruct((b, c), jnp.float32),
            jax.ShapeDtypeStruct((b, 1), jnp.float32),
        ],
    )(logits_w, partial_y)


def _sinkhorn_body(queue_ref, cost_ref, emp_ref, out_ref,
                   u_ref, r_ref, clast_ref, scal_ref, *, nq, tc, n_total):
    p = pl.program_id(0)
    t = pl.program_id(1)
    inv_n = 1.0 / n_total

    emp_raw = emp_ref[0:1, :]
    emp_row = emp_raw / jnp.sum(emp_raw)

    @pl.when((p >= 1) & (p <= N_SINKHORN) & (t == 0))
    def _():
        u = u_ref[0:1, :] + EPS * scal_ref[0]
        r = emp_row / u
        r_ref[0:1, :] = r
        scal_ref[1] = jnp.sum(r)

    @pl.when((p <= N_SINKHORN) & (t == 0))
    def _():
        u_ref[0:1, :] = jnp.zeros_like(u_ref[0:1, :])
        scal_ref[0] = jnp.where(p == 0, 1.0, 0.0)

    def tile_work(tile, is_cost):
        rows = tile.shape[0]

        @pl.when(p == 0)
        def _():
            c_bc = jnp.full((rows, 128), inv_n, jnp.float32)
            u_part = jax.lax.dot_general(c_bc, tile, (((0,), (0,)), ((), ())))
            u_ref[0:1, :] += u_part[0:1, :]

        @pl.when((p >= 1) & (p <= N_SINKHORN))
        def _():
            r_bc = jnp.broadcast_to(r_ref[0:1, :], (128, tile.shape[1]))
            w_full = jax.lax.dot_general(tile, r_bc, (((1,), (1,)), ((), ())))
            w = w_full[:, 0:1] + EPS * scal_ref[1]
            c_t = inv_n / w
            c_bc = jnp.broadcast_to(c_t, (rows, 128))
            u_part = jax.lax.dot_general(c_bc, tile, (((0,), (0,)), ((), ())))
            u_ref[0:1, :] += u_part[0:1, :]
            scal_ref[0] += jnp.sum(c_t)

            if is_cost:
                @pl.when(p == N_SINKHORN)
                def _():
                    clast_ref[pl.ds((t - nq) * tc, tc), :] = c_t

    @pl.when((t < nq) & (p <= N_SINKHORN))
    def _():
        tile_work(queue_ref[:, :], False)

    @pl.when(t >= nq)
    def _():
        tile_work(cost_ref[:, :], True)

    @pl.when((p == N_SINKHORN + 1) & (t >= nq))
    def _():
        u = u_ref[0:1, :] + EPS * scal_ref[0]
        r_fin = emp_row / u
        c_blk = clast_ref[pl.ds((t - nq) * tc, tc), :]
        out_ref[:, :] = ((cost_ref[:, :] + EPS) * c_blk
                         * r_fin * float(n_total))


def _sinkhorn(queue, cost, emp_bc, tq=1024):
    q, c = queue.shape
    b = cost.shape[0]
    tc = min(tq, b)
    nq = q // tq
    nb = b // tc
    n_total = q + b
    grid = (N_SINKHORN + 2, nq + nb)

    def qmap(p, t):
        return (jnp.where(p > N_SINKHORN, nq - 1, jnp.minimum(t, nq - 1)), 0)

    def cmap(p, t):
        return (jnp.where(t < nq, nb - 1, t - nq), 0)

    def omap(p, t):
        return (jnp.where(p <= N_SINKHORN, 0,
                          jnp.clip(t - nq, 0, nb - 1)), 0)

    return pl.pallas_call(
        functools.partial(_sinkhorn_body, nq=nq, tc=tc, n_total=n_total),
        grid=grid,
        in_specs=[
            pl.BlockSpec((tq, c), qmap),
            pl.BlockSpec((tc, c), cmap),
            pl.BlockSpec((8, c), lambda p, t: (0, 0)),
        ],
        out_specs=pl.BlockSpec((tc, c), omap),
        out_shape=jax.ShapeDtypeStruct((b, c), jnp.float32),
        scratch_shapes=[
            pltpu.VMEM((1, c), jnp.float32),
            pltpu.VMEM((1, c), jnp.float32),
            pltpu.VMEM((b, 1), jnp.float32),
            pltpu.SMEM((4,), jnp.float32),
        ],
    )(queue, cost, emp_bc)


def _rowstats_body(plabel_ref, logsm_ref, pred_ref, emp_ref,
                   ploss_ref, pidx_ref, hc_ref, kc_ref, *, bsz):
    plab = plabel_ref[:, :]
    tb, c = plab.shape
    emp_raw = emp_ref[0:1, :]
    emp_row = emp_raw / jnp.sum(emp_raw)

    ploss_ref[:, :] = -jnp.sum(logsm_ref[:, :] * plab, axis=1, keepdims=True)

    iota = jax.lax.broadcasted_iota(jnp.int32, (tb, c), 1)
    mx = jnp.max(plab, axis=1, keepdims=True)
    idx = jnp.min(jnp.where(plab == mx, iota, c), axis=1, keepdims=True)
    pidx_ref[:, :] = idx

    metric = jnp.sum(plab * pred_ref[:, :], axis=1, keepdims=True)
    hc_ref[:, :] = jnp.where(metric > TAU, 1.0, 0.0)

    emp_at = jnp.sum(jnp.where(iota == idx, emp_row, 0.0),
                     axis=1, keepdims=True)
    kc_ref[:, :] = jnp.ceil(float(bsz) * emp_at * RHO).astype(jnp.int32)


def _rowstats(plabel, logsm, pred, emp_bc, tb=512):
    b, c = plabel.shape
    grid = (b // tb,)
    bs = pl.BlockSpec((tb, c), lambda i: (i, 0))
    vs = pl.BlockSpec((tb, 1), lambda i: (i, 0))
    return pl.pallas_call(
        functools.partial(_rowstats_body, bsz=b),
        grid=grid,
        in_specs=[bs, bs, bs, pl.BlockSpec((8, c), lambda i: (0, 0))],
        out_specs=[vs, vs, vs, vs],
        out_shape=[
            jax.ShapeDtypeStruct((b, 1), jnp.float32),
            jax.ShapeDtypeStruct((b, 1), jnp.int32),
            jax.ShapeDtypeStruct((b, 1), jnp.float32),
            jax.ShapeDtypeStruct((b, 1), jnp.int32),
        ],
    )(plabel, logsm, pred, emp_bc)


def _select_body(pidx_ref, ploss_ref, kc_ref, hc_ref, rn_ref,
                 pidxT_ref, plossT_ref, out_ref, *, tb):
    t = pl.program_id(0)
    idx_col = pidx_ref[:, :]
    loss_col = ploss_ref[:, :]
    n = pidxT_ref.shape[1]
    idx_row = pidxT_ref[0:1, :]
    loss_row = plossT_ref[0:1, :]

    row_ids = t * tb + jax.lax.broadcasted_iota(jnp.int32, (tb, 1), 0)
    col_ids = jax.lax.broadcasted_iota(jnp.int32, (tb, n), 1)

    eq = idx_row == idx_col
    before = (loss_row < loss_col) | ((loss_row == loss_col)
                                      & (col_ids < row_ids))
    rank = jnp.sum(jnp.where(eq & before, 1.0, 0.0), axis=1, keepdims=True)
    count = jnp.sum(jnp.where(eq, 1.0, 0.0), axis=1, keepdims=True)

    kf = kc_ref[:, :].astype(jnp.float32)
    k_eff = jnp.minimum(jnp.maximum(kf, 1.0), jnp.maximum(count, 1.0))
    sel = (rank < k_eff) | (hc_ref[:, :] > 0.5)
    out_ref[:, :] = jnp.where(sel, loss_col, rn_ref[:, :])


def _select(pidx, ploss, kc, hc, rn, tb=512):
    b = pidx.shape[0]
    pidxT = jnp.broadcast_to(pidx.reshape(1, b), (8, b))
    plossT = jnp.broadcast_to(ploss.reshape(1, b), (8, b))
    grid = (b // tb,)
    vs = pl.BlockSpec((tb, 1), lambda i: (i, 0))
    ts = pl.BlockSpec((8, b), lambda i: (0, 0))
    return pl.pallas_call(
        functools.partial(_select_body, tb=tb),
        grid=grid,
        in_specs=[vs, vs, vs, vs, vs, ts, ts],
        out_specs=vs,
        out_shape=jax.ShapeDtypeStruct((b, 1), jnp.float32),
    )(pidx, ploss, kc, hc, rn, pidxT, plossT)


def kernel(logits_w, partial_y, queue, emp_dist):
    b, c = logits_w.shape
    emp_bc = jnp.broadcast_to(emp_dist.reshape(1, c), (8, c))
    pred, cost, logsm, rn = _prologue(logits_w, partial_y)
    plabel = _sinkhorn(queue, cost, emp_bc)
    ploss, pidx, hc, kc = _rowstats(plabel, logsm, pred, emp_bc)
    out = _select(pidx, ploss, kc, hc, rn)
    return out.reshape(b)
